# Initial kernel scaffold; baseline (speedup 1.0000x reference)
import jax
import jax.numpy as jnp
from jax.experimental import pallas as pl


def kernel(boxes_t, t, feat0, feat1, feat2, feat3, w1, b1, wt, bt, w2, b2, wreg, breg, wcls, bcls):
    raise NotImplementedError("write your pallas kernel here")


# speedup vs baseline: 19.7778x; 19.7778x over previous
import math

import jax
import jax.numpy as jnp
from jax.experimental import pallas as pl
from jax.experimental.pallas import tpu as pltpu

R = 7
SR = 2
S = R * SR
SPAD = 16
T_DIM = 128
NUM_CLASSES = 80
B, N, C = 4, 256, 256
K = B * N
HIDDEN = 1024
IN_DIM = C * R * R
LVL_H = (256, 128, 64, 32)
LVL_W = (256, 128, 64, 32)
LVL_OFF = (0, 65536, 81920, 86016)
NPIX = 86016 + 32 * 32
NPIX_PAD = NPIX + 64
SCALES = (0.25, 0.125, 0.0625, 0.03125)
KBLK = 7
KCHUNK = IN_DIM // KBLK


def _axis_tables(coord, size):
    sizef = size.astype(jnp.float32)[:, None]
    valid = (coord >= -1.0) & (coord <= sizef)
    c = jnp.maximum(coord, 0.0)
    low = jnp.floor(c)
    low_i = jnp.minimum(low, sizef - 1.0)
    c = jnp.where(low >= sizef - 1.0, sizef - 1.0, c)
    frac = c - low_i
    v = valid.astype(jnp.float32)
    return low_i.astype(jnp.int32), (1.0 - frac) * v, frac * v


def _gather_body(rlo_s, rhi_s, wyl_s, wyh_s, xc_s, wxl_s, wxh_s,
                 feats_hbm, out_ref, fvm, sem):
    b = pl.program_id(0) * 2 + pl.program_id(1)
    i = pl.program_id(2)

    @pl.when(i == 0)
    def _():
        cp = pltpu.make_async_copy(feats_hbm.at[b], fvm, sem)
        cp.start()
        cp.wait()

    k16 = (b * N + i) * SPAD

    def by_body(by, carry):
        rls, rhs, wls, whs = [], [], [], []
        for sy in range(2):
            iyi = k16 + 2 * by + sy
            rls.append(rlo_s[iyi])
            rhs.append(rhi_s[iyi])
            wls.append(wyl_s[iyi])
            whs.append(wyh_s[iyi])
        pieces = []
        for bx in range(R):
            acc = None
            for sy in range(2):
                rl, rh, wl, wh = rls[sy], rhs[sy], wls[sy], whs[sy]
                for sx in range(2):
                    ix = k16 + 2 * bx + sx
                    x = xc_s[ix]
                    al = wxl_s[ix]
                    ah = wxh_s[ix]
                    a32 = pltpu.bitcast(fvm[rl + x], jnp.bfloat16).astype(jnp.float32)
                    b32 = pltpu.bitcast(fvm[rl + x + 1], jnp.bfloat16).astype(jnp.float32)
                    c32 = pltpu.bitcast(fvm[rh + x], jnp.bfloat16).astype(jnp.float32)
                    d32 = pltpu.bitcast(fvm[rh + x + 1], jnp.bfloat16).astype(jnp.float32)
                    smp = wl * (al * a32 + ah * b32) + wh * (al * c32 + ah * d32)
                    acc = smp if acc is None else acc + smp
            pieces.append(acc)
        out_ref[by] = jnp.concatenate(pieces, axis=0)
        return carry

    jax.lax.fori_loop(0, R, by_body, 0)


def _mlp_body(x_ref, w1_ref, e_ref, wt_ref, b1_ref, bt_ref,
              w2_ref, b2_ref, wh_ref, bh_ref, out_ref, acc_ref):
    j = pl.program_id(2)

    @pl.when(j == 0)
    def _():
        acc_ref[...] = jnp.zeros_like(acc_ref)

    acc_ref[...] += jnp.dot(x_ref[...], w1_ref[...],
                            preferred_element_type=jnp.float32)

    @pl.when(j == KBLK - 1)
    def _():
        h1 = jnp.maximum(acc_ref[...] + b1_ref[...], 0.0)
        te = jnp.maximum(jnp.dot(e_ref[...], wt_ref[...],
                                 preferred_element_type=jnp.float32)
                         + bt_ref[...], 0.0)
        h1 = h1 + te
        h2 = jnp.maximum(jnp.dot(h1, w2_ref[...],
                                 preferred_element_type=jnp.float32)
                         + b2_ref[...], 0.0)
        out_ref[...] = (jnp.dot(h2, wh_ref[...],
                                preferred_element_type=jnp.float32)
                        + bh_ref[...])


def kernel(boxes_t, t, feat0, feat1, feat2, feat3,
           w1, b1, wt, bt, w2, b2, wreg, breg, wcls, bcls):
    f32 = jnp.float32

    cx, cy, w, h = (boxes_t[..., i] for i in range(4))
    x1 = (cx - 0.5 * w).reshape(K)
    y1 = (cy - 0.5 * h).reshape(K)
    x2 = (cx + 0.5 * w).reshape(K)
    y2 = (cy + 0.5 * h).reshape(K)
    area = ((x2 - x1) * (y2 - y1))
    lvl = jnp.floor(4.0 + jnp.log2(jnp.sqrt(area) / 224.0 + 1e-6))
    idx = (jnp.clip(lvl, 2.0, 5.0) - 2.0).astype(jnp.int32)

    scale = jnp.take(jnp.asarray(SCALES, f32), idx)
    Wl = jnp.take(jnp.asarray(LVL_W, jnp.int32), idx)
    Hl = jnp.take(jnp.asarray(LVL_H, jnp.int32), idx)
    off = jnp.take(jnp.asarray(LVL_OFF, jnp.int32), idx)

    x1s = x1 * scale
    y1s = y1 * scale
    bw = jnp.maximum(x2 * scale - x1s, 1.0) / R
    bh = jnp.maximum(y2 * scale - y1s, 1.0) / R
    grid = (jnp.arange(S, dtype=f32) + 0.5) / SR
    ys = y1s[:, None] + grid[None, :] * bh[:, None]
    xs = x1s[:, None] + grid[None, :] * bw[:, None]

    yl, wyl, wyh = _axis_tables(ys, Hl)
    xl, wxl, wxh = _axis_tables(xs, Wl)
    wyl = wyl * 0.25
    wyh = wyh * 0.25

    rlo = off[:, None] + yl * Wl[:, None] + 0
    rhi = rlo + Wl[:, None]

    def padflat(a):
        return jnp.pad(a, ((0, 0), (0, SPAD - S))).reshape(K * SPAD)

    t_rlo = padflat(rlo)
    t_rhi = padflat(rhi)
    t_wyl = padflat(wyl)
    t_wyh = padflat(wyh)
    t_xc = padflat(xl)
    t_wxl = padflat(wxl)
    t_wxh = padflat(wxh)

    def chlast(feat):
        Bb, Cc, Hh, Ww = feat.shape
        return feat.astype(jnp.bfloat16).transpose(0, 2, 3, 1).reshape(Bb, Hh * Ww, Cc)

    allf = jnp.concatenate(
        [chlast(feat0), chlast(feat1), chlast(feat2), chlast(feat3)], axis=1)
    allf = jnp.pad(allf, ((0, 0), (0, NPIX_PAD - NPIX), (0, 0)))
    packed = jax.lax.bitcast_convert_type(
        allf.reshape(B, NPIX_PAD, 1, 2, 128).transpose(0, 1, 2, 4, 3),
        jnp.int32).reshape(B, NPIX_PAD, 1, 128)

    crops = pl.pallas_call(
        _gather_body,
        out_shape=jax.ShapeDtypeStruct((B, N, R, S, 128), f32),
        grid_spec=pltpu.PrefetchScalarGridSpec(
            num_scalar_prefetch=7,
            grid=(2, 2, N),
            in_specs=[pl.BlockSpec(memory_space=pl.ANY)],
            out_specs=pl.BlockSpec((None, None, R, S, 128),
                                   lambda c, bb, i, *_: (c * 2 + bb, i, 0, 0, 0)),
            scratch_shapes=[pltpu.VMEM((NPIX_PAD, 1, 128), jnp.int32),
                            pltpu.SemaphoreType.DMA],
        ),
        compiler_params=pltpu.CompilerParams(
            dimension_semantics=("parallel", "parallel", "arbitrary"),
            vmem_limit_bytes=56 * 1024 * 1024,
        ),
        name="roialign_gather",
    )(t_rlo, t_rhi, t_wyl, t_wyh, t_xc, t_wxl, t_wxh, packed)

    x = crops.reshape(K, IN_DIM)

    w1p = w1.reshape(C, R, R, HIDDEN).transpose(1, 2, 0, 3).reshape(IN_DIM, HIDDEN)
    half = T_DIM // 2
    freqs = jnp.exp(-math.log(10000.0) * jnp.arange(half, dtype=f32) / half)
    ang = t[:, None] * freqs
    emb = jnp.concatenate([jnp.sin(ang), jnp.cos(ang)], axis=-1).reshape(B, 1, T_DIM)

    wh_pack = jnp.pad(jnp.concatenate([wreg, wcls], axis=1), ((0, 0), (0, 44)))
    bh_pack = jnp.pad(jnp.concatenate([breg, bcls])[None, :], ((0, 0), (0, 44)))

    out = pl.pallas_call(
        _mlp_body,
        out_shape=jax.ShapeDtypeStruct((K, 128), f32),
        grid=(2, 2, KBLK),
        in_specs=[
            pl.BlockSpec((N, KCHUNK), lambda c, ii, j: (c * 2 + ii, j)),
            pl.BlockSpec((KCHUNK, HIDDEN), lambda c, ii, j: (j, 0)),
            pl.BlockSpec((None, 1, T_DIM), lambda c, ii, j: (c * 2 + ii, 0, 0)),
            pl.BlockSpec((T_DIM, HIDDEN), lambda c, ii, j: (0, 0)),
            pl.BlockSpec((1, HIDDEN), lambda c, ii, j: (0, 0)),
            pl.BlockSpec((1, HIDDEN), lambda c, ii, j: (0, 0)),
            pl.BlockSpec((HIDDEN, HIDDEN), lambda c, ii, j: (0, 0)),
            pl.BlockSpec((1, HIDDEN), lambda c, ii, j: (0, 0)),
            pl.BlockSpec((HIDDEN, 128), lambda c, ii, j: (0, 0)),
            pl.BlockSpec((1, 128), lambda c, ii, j: (0, 0)),
        ],
        out_specs=pl.BlockSpec((N, 128), lambda c, ii, j: (c * 2 + ii, 0)),
        scratch_shapes=[pltpu.VMEM((N, HIDDEN), f32)],
        compiler_params=pltpu.CompilerParams(
            dimension_semantics=("parallel", "parallel", "arbitrary"),
            vmem_limit_bytes=56 * 1024 * 1024,
        ),
        name="decoder_mlp",
    )(x, w1p, emb, wt, b1.reshape(1, HIDDEN), bt.reshape(1, HIDDEN),
      w2, b2.reshape(1, HIDDEN), wh_pack, bh_pack)

    delta = out[:, :4].reshape(B, N, 4)
    logits = out[:, 4:4 + NUM_CLASSES].reshape(B, N, NUM_CLASSES)
    return boxes_t + delta, logits
